# baseline (speedup 1.0000x reference)
import functools
import math

import jax
import jax.numpy as jnp
import numpy as np
from jax import lax
from jax.experimental import pallas as pl
from jax.experimental.pallas import tpu as pltpu
from jax.experimental.pallas import tpu_sc as plsc

_ALPHA = 0.1
_THETA = 0.5

_NC = 2
_NS = 16
_NW = _NC * _NS
_CHUNK = 128


def _sc_mesh():
    return plsc.VectorSubcoreMesh(
        core_axis_name="c", subcore_axis_name="s",
        num_cores=_NC, num_subcores=_NS)


def _pad_rows(n):
    per = -(-n // _NS)
    per = -(-per // 8) * 8
    return per * _NS, per


@functools.partial(jax.jit, static_argnums=(1,))
def _sc_degree(dst, n):
    e = dst.shape[0]
    ew = e // _NW
    nfull = ew // _CHUNK
    tail = ew % _CHUNK
    n_pad, rows_per_tile = _pad_rows(n)
    dcol = 128

    ones_rows = jnp.ones((_CHUNK, dcol), jnp.float32)
    zero_rows = jnp.zeros((_CHUNK, dcol), jnp.float32)

    @functools.partial(
        pl.kernel,
        out_type=jax.ShapeDtypeStruct((_NC, n_pad, dcol), jnp.float32),
        mesh=_sc_mesh(),
        scratch_types=[
            pltpu.VMEM((_CHUNK,), jnp.int32),
            pltpu.VMEM((tail,), jnp.int32),
            pltpu.VMEM((_CHUNK, dcol), jnp.float32),
            pltpu.VMEM((_CHUNK, dcol), jnp.float32),
            pltpu.VMEM_SHARED((n_pad, dcol), jnp.float32),
        ],
    )
    def k(dst_hbm, ones_hbm, zeros_hbm, out_hbm, dst_v, dstt_v, ones_v,
          zeros_v, acc_sh):
        cid = lax.axis_index("c")
        sid = lax.axis_index("s")
        wid = cid * _NS + sid
        base = wid * ew
        r0 = sid * rows_per_tile

        pltpu.sync_copy(ones_hbm, ones_v)
        pltpu.sync_copy(zeros_hbm, zeros_v)
        zc = _CHUNK
        for j in range((rows_per_tile + zc - 1) // zc):
            m = min(zc, rows_per_tile - j * zc)
            pltpu.sync_copy(zeros_v.at[pl.ds(0, m)],
                            acc_sh.at[pl.ds(r0 + j * zc, m)])
        plsc.subcore_barrier()

        @pl.loop(0, nfull)
        def body(i):
            eb = base + i * _CHUNK
            pltpu.sync_copy(dst_hbm.at[pl.ds(eb, _CHUNK)], dst_v)
            pltpu.sync_copy(ones_v, acc_sh.at[dst_v], add=True)

        if tail:
            pltpu.sync_copy(dst_hbm.at[pl.ds(base + nfull * _CHUNK, tail)],
                            dstt_v)
            pltpu.sync_copy(ones_v.at[pl.ds(0, tail)], acc_sh.at[dstt_v],
                            add=True)
        plsc.subcore_barrier()
        pltpu.sync_copy(acc_sh.at[pl.ds(r0, rows_per_tile)],
                        out_hbm.at[cid, pl.ds(r0, rows_per_tile)])

    return k(dst, ones_rows, zero_rows)


_PH = 40


def _pad_edges(src, dst, n):
    e = src.shape[0]
    ew = e // _NW
    ewp = -(-ew // (_PH * _CHUNK)) * (_PH * _CHUNK)
    nch = ewp // _CHUNK
    srcw = src.reshape(_NW, ew)
    dstw = dst.reshape(_NW, ew)
    if ewp != ew:
        pad_s = jnp.zeros((_NW, ewp - ew), jnp.int32)
        pad_d = jnp.full((_NW, ewp - ew), n, jnp.int32)
        srcw = jnp.concatenate([srcw, pad_s], axis=1)
        dstw = jnp.concatenate([dstw, pad_d], axis=1)
    return (srcw.reshape(_NW, nch, _CHUNK), dstw.reshape(_NW, nch, _CHUNK),
            nch)


@functools.partial(jax.jit, static_argnums=(3,))
def _sc_scatter(table, src3, dst3, nch):
    n, d = table.shape
    n_pad, rows_per_tile = _pad_rows(n)

    zero_rows = jnp.zeros((_CHUNK, d), jnp.float32)

    @functools.partial(
        pl.kernel,
        out_type=jax.ShapeDtypeStruct((_NC, n_pad, d), jnp.float32),
        mesh=_sc_mesh(),
        scratch_types=[
            pltpu.VMEM((_PH, _CHUNK), jnp.int32),
            pltpu.VMEM((_PH, _CHUNK), jnp.int32),
            pltpu.VMEM((_CHUNK, d), jnp.float32),
            pltpu.VMEM((_CHUNK, d), jnp.float32),
            pltpu.VMEM((_CHUNK,), jnp.int32),
            pltpu.VMEM((_CHUNK,), jnp.int32),
            pltpu.VMEM((_CHUNK,), jnp.int32),
            pltpu.VMEM_SHARED((n_pad, d), jnp.float32),
            pltpu.SemaphoreType.DMA,
            pltpu.SemaphoreType.DMA,
        ],
    )
    def k(table_hbm, src_hbm, dst_hbm, zeros_hbm, out_hbm,
          src_v, dst_v, rows0, rows1, sidx0, sidx1, didx, acc_sh,
          sem0, sem1):
        cid = lax.axis_index("c")
        sid = lax.axis_index("s")
        wid = cid * _NS + sid
        r0 = sid * rows_per_tile

        def stage_idx(buf, slab, j):
            for kk in range(_CHUNK // 16):
                buf[pl.ds(kk * 16, 16)] = slab[j, pl.ds(kk * 16, 16)]

        pltpu.sync_copy(zeros_hbm, rows0)
        for j in range((rows_per_tile + _CHUNK - 1) // _CHUNK):
            m = min(_CHUNK, rows_per_tile - j * _CHUNK)
            pltpu.sync_copy(rows0.at[pl.ds(0, m)],
                            acc_sh.at[pl.ds(r0 + j * _CHUNK, m)])
        plsc.subcore_barrier()

        for p in range(nch // _PH):
            pltpu.sync_copy(src_hbm.at[wid, pl.ds(p * _PH, _PH)], src_v)
            pltpu.sync_copy(dst_hbm.at[wid, pl.ds(p * _PH, _PH)], dst_v)

            @pl.loop(0, _PH)
            def body(j):
                pltpu.async_copy(table_hbm.at[src_v.at[j]], rows0,
                                 sem0).wait()
                pltpu.sync_copy(rows0, acc_sh.at[dst_v.at[j]], add=True)

        plsc.subcore_barrier()
        pltpu.sync_copy(acc_sh.at[pl.ds(r0, rows_per_tile)],
                        out_hbm.at[cid, pl.ds(r0, rows_per_tile)])

    return k(table, src3, dst3, zero_rows)


_BLK = 1000


def _dinv_from_degp(degp):
    deg = degp[0, :, 0] + degp[1, :, 0] + 1.0
    return lax.rsqrt(deg)


def _pre_body(degp_ref, x_ref, w1_ref, b1_ref, h_ref, hs_ref):
    dinv = _dinv_from_degp(degp_ref[...])
    h = jnp.maximum(
        jnp.dot(x_ref[...], w1_ref[...],
                preferred_element_type=jnp.float32) + b1_ref[...], 0.0)
    h_ref[...] = h
    hs_ref[...] = h * dinv[:, None]


def _pre(degp, x, w1, b1):
    n, dft = x.shape
    hid = w1.shape[1]
    grid = n // _BLK
    return pl.pallas_call(
        _pre_body,
        grid=(grid,),
        in_specs=[
            pl.BlockSpec((_NC, _BLK, 128), lambda i: (0, i, 0)),
            pl.BlockSpec((_BLK, dft), lambda i: (i, 0)),
            pl.BlockSpec((dft, hid), lambda i: (0, 0)),
            pl.BlockSpec((1, hid), lambda i: (0, 0)),
        ],
        out_specs=[
            pl.BlockSpec((_BLK, hid), lambda i: (i, 0)),
            pl.BlockSpec((_BLK, hid), lambda i: (i, 0)),
        ],
        out_shape=[
            jax.ShapeDtypeStruct((n, hid), jnp.float32),
            jax.ShapeDtypeStruct((n, hid), jnp.float32),
        ],
    )(degp, x, w1, b1.reshape(1, hid))


def _layer_body(beta, degp_ref, part_ref, curs_ref, x0_ref, w_ref,
                cur_ref, curs_out_ref):
    dinv = _dinv_from_degp(degp_ref[...])
    s = part_ref[0] + part_ref[1] + curs_ref[...]
    agg = s * dinv[:, None]
    out = (1.0 - _ALPHA) * agg + _ALPHA * x0_ref[...]
    out = (1.0 - beta) * out + beta * jnp.dot(
        out, w_ref[...], preferred_element_type=jnp.float32)
    cur = jnp.maximum(out, 0.0)
    cur_ref[...] = cur
    curs_out_ref[...] = cur * dinv[:, None]


def _layer(beta, degp, part, curs, x0, w):
    n, hid = x0.shape
    grid = n // _BLK
    return pl.pallas_call(
        functools.partial(_layer_body, beta),
        grid=(grid,),
        in_specs=[
            pl.BlockSpec((_NC, _BLK, 128), lambda i: (0, i, 0)),
            pl.BlockSpec((_NC, _BLK, hid), lambda i: (0, i, 0)),
            pl.BlockSpec((_BLK, hid), lambda i: (i, 0)),
            pl.BlockSpec((_BLK, hid), lambda i: (i, 0)),
            pl.BlockSpec((hid, hid), lambda i: (0, 0)),
        ],
        out_specs=[
            pl.BlockSpec((_BLK, hid), lambda i: (i, 0)),
            pl.BlockSpec((_BLK, hid), lambda i: (i, 0)),
        ],
        out_shape=[
            jax.ShapeDtypeStruct((n, hid), jnp.float32),
            jax.ShapeDtypeStruct((n, hid), jnp.float32),
        ],
    )(degp, part, curs, x0, w)


def _final_body(cur_ref, w2_ref, b2_ref, o_ref):
    logits = jnp.dot(cur_ref[...], w2_ref[...],
                     preferred_element_type=jnp.float32) + b2_ref[...]
    m = jnp.max(logits, axis=1, keepdims=True)
    shifted = logits - m
    lse = jnp.log(jnp.sum(jnp.exp(shifted), axis=1, keepdims=True))
    o_ref[...] = shifted - lse


def _final(cur, w2, b2):
    n, hid = cur.shape
    nc = w2.shape[1]
    grid = n // _BLK
    return pl.pallas_call(
        _final_body,
        grid=(grid,),
        in_specs=[
            pl.BlockSpec((_BLK, hid), lambda i: (i, 0)),
            pl.BlockSpec((hid, nc), lambda i: (0, 0)),
            pl.BlockSpec((1, nc), lambda i: (0, 0)),
        ],
        out_specs=pl.BlockSpec((_BLK, nc), lambda i: (i, 0)),
        out_shape=jax.ShapeDtypeStruct((n, nc), jnp.float32),
    )(cur, w2, b2.reshape(1, nc))


def kernel(x, edge_index, y, W1, b1, Ws, W2, b2):
    n = x.shape[0]
    src = edge_index[0]
    dst = edge_index[1]
    src3, dst3, nch = _pad_edges(src, dst, n)

    degp = _sc_degree(dst, n)
    h, curs = _pre(degp, x, W1, b1)
    x0 = h
    cur = h
    for i in range(Ws.shape[0]):
        part = _sc_scatter(curs, src3, dst3, nch)
        beta = float(np.log(_THETA / (i + 1) + 1.0))
        cur, curs = _layer(beta, degp, part, curs, x0, Ws[i])
    return _final(cur, W2, b2)


# speedup vs baseline: 1.7088x; 1.7088x over previous
import functools
import math

import jax
import jax.numpy as jnp
import numpy as np
from jax import lax
from jax.experimental import pallas as pl
from jax.experimental.pallas import tpu as pltpu
from jax.experimental.pallas import tpu_sc as plsc

_ALPHA = 0.1
_THETA = 0.5

_NC = 2
_NS = 16
_NW = _NC * _NS
_CHUNK = 128


def _sc_mesh():
    return plsc.VectorSubcoreMesh(
        core_axis_name="c", subcore_axis_name="s",
        num_cores=_NC, num_subcores=_NS)


def _pad_rows(n):
    per = -(-n // _NS)
    per = -(-per // 8) * 8
    return per * _NS, per


@functools.partial(jax.jit, static_argnums=(1,))
def _sc_degree(dst, n):
    e = dst.shape[0]
    ew = e // _NW
    nfull = ew // _CHUNK
    tail = ew % _CHUNK
    n_pad, rows_per_tile = _pad_rows(n)
    dcol = 128

    ones_rows = jnp.ones((_CHUNK, dcol), jnp.float32)
    zero_rows = jnp.zeros((_CHUNK, dcol), jnp.float32)

    @functools.partial(
        pl.kernel,
        out_type=jax.ShapeDtypeStruct((_NC, n_pad, dcol), jnp.float32),
        mesh=_sc_mesh(),
        scratch_types=[
            pltpu.VMEM((_CHUNK,), jnp.int32),
            pltpu.VMEM((tail,), jnp.int32),
            pltpu.VMEM((_CHUNK, dcol), jnp.float32),
            pltpu.VMEM((_CHUNK, dcol), jnp.float32),
            pltpu.VMEM_SHARED((n_pad, dcol), jnp.float32),
        ],
    )
    def k(dst_hbm, ones_hbm, zeros_hbm, out_hbm, dst_v, dstt_v, ones_v,
          zeros_v, acc_sh):
        cid = lax.axis_index("c")
        sid = lax.axis_index("s")
        wid = cid * _NS + sid
        base = wid * ew
        r0 = sid * rows_per_tile

        pltpu.sync_copy(ones_hbm, ones_v)
        pltpu.sync_copy(zeros_hbm, zeros_v)
        zc = _CHUNK
        for j in range((rows_per_tile + zc - 1) // zc):
            m = min(zc, rows_per_tile - j * zc)
            pltpu.sync_copy(zeros_v.at[pl.ds(0, m)],
                            acc_sh.at[pl.ds(r0 + j * zc, m)])
        plsc.subcore_barrier()

        @pl.loop(0, nfull)
        def body(i):
            eb = base + i * _CHUNK
            pltpu.sync_copy(dst_hbm.at[pl.ds(eb, _CHUNK)], dst_v)
            pltpu.sync_copy(ones_v, acc_sh.at[dst_v], add=True)

        if tail:
            pltpu.sync_copy(dst_hbm.at[pl.ds(base + nfull * _CHUNK, tail)],
                            dstt_v)
            pltpu.sync_copy(ones_v.at[pl.ds(0, tail)], acc_sh.at[dstt_v],
                            add=True)
        plsc.subcore_barrier()
        pltpu.sync_copy(acc_sh.at[pl.ds(r0, rows_per_tile)],
                        out_hbm.at[cid, pl.ds(r0, rows_per_tile)])

    return k(dst, ones_rows, zero_rows)


def _sc_scatter_build(n, d, e):
    ew = e // _NW
    nfull = ew // _CHUNK
    tail = ew % _CHUNK
    n_pad, rows_per_tile = _pad_rows(n)

    @functools.partial(
        pl.kernel,
        out_type=jax.ShapeDtypeStruct((_NC, n_pad, d), jnp.float32),
        mesh=_sc_mesh(),
        scratch_types=[
            pltpu.VMEM((_CHUNK,), jnp.int32),
            pltpu.VMEM((_CHUNK,), jnp.int32),
            pltpu.VMEM((tail,), jnp.int32),
            pltpu.VMEM((tail,), jnp.int32),
            pltpu.VMEM((_CHUNK, d), jnp.float32),
            pltpu.VMEM((tail, d), jnp.float32),
            pltpu.VMEM_SHARED((n_pad, d), jnp.float32),
            pltpu.SemaphoreType.DMA,
        ],
    )
    def k(table_hbm, src_hbm, dst_hbm, zeros_hbm, out_hbm,
          src_v, dst_v, srct_v, dstt_v, rows_v, rowst_v, acc_sh, sem):
        cid = lax.axis_index("c")
        sid = lax.axis_index("s")
        wid = cid * _NS + sid
        base = wid * ew
        r0 = sid * rows_per_tile

        pltpu.sync_copy(zeros_hbm, rows_v)
        zc = _CHUNK
        for j in range((rows_per_tile + zc - 1) // zc):
            m = min(zc, rows_per_tile - j * zc)
            pltpu.sync_copy(rows_v.at[pl.ds(0, m)],
                            acc_sh.at[pl.ds(r0 + j * zc, m)])
        plsc.subcore_barrier()

        @pl.loop(0, nfull)
        def body(i):
            eb = base + i * _CHUNK
            pltpu.sync_copy(src_hbm.at[pl.ds(eb, _CHUNK)], src_v)
            pltpu.sync_copy(dst_hbm.at[pl.ds(eb, _CHUNK)], dst_v)
            pltpu.async_copy(table_hbm.at[src_v], rows_v, sem).wait()
            pltpu.sync_copy(rows_v, acc_sh.at[dst_v], add=True)

        if tail:
            eb = base + nfull * _CHUNK
            pltpu.sync_copy(src_hbm.at[pl.ds(eb, tail)], srct_v)
            pltpu.sync_copy(dst_hbm.at[pl.ds(eb, tail)], dstt_v)
            pltpu.async_copy(table_hbm.at[srct_v], rowst_v, sem).wait()
            pltpu.sync_copy(rowst_v, acc_sh.at[dstt_v], add=True)
        plsc.subcore_barrier()
        pltpu.sync_copy(acc_sh.at[pl.ds(r0, rows_per_tile)],
                        out_hbm.at[cid, pl.ds(r0, rows_per_tile)])

    return k


@functools.partial(jax.jit, static_argnums=())
def _sc_scatter(table, src, dst):
    n, d = table.shape
    zero_rows = jnp.zeros((_CHUNK, d), jnp.float32)
    k = _sc_scatter_build(n, d, src.shape[0])
    return k(table, src, dst, zero_rows)


_BLK = 1000


def _dinv_from_degp(degp):
    deg = degp[0, :, 0] + degp[1, :, 0] + 1.0
    return lax.rsqrt(deg)


def _pre_body(degp_ref, x_ref, w1_ref, b1_ref, h_ref, hs_ref):
    dinv = _dinv_from_degp(degp_ref[...])
    h = jnp.maximum(
        jnp.dot(x_ref[...], w1_ref[...],
                preferred_element_type=jnp.float32) + b1_ref[...], 0.0)
    h_ref[...] = h
    hs_ref[...] = h * dinv[:, None]


def _pre(degp, x, w1, b1):
    n, dft = x.shape
    hid = w1.shape[1]
    grid = n // _BLK
    return pl.pallas_call(
        _pre_body,
        grid=(grid,),
        in_specs=[
            pl.BlockSpec((_NC, _BLK, 128), lambda i: (0, i, 0)),
            pl.BlockSpec((_BLK, dft), lambda i: (i, 0)),
            pl.BlockSpec((dft, hid), lambda i: (0, 0)),
            pl.BlockSpec((1, hid), lambda i: (0, 0)),
        ],
        out_specs=[
            pl.BlockSpec((_BLK, hid), lambda i: (i, 0)),
            pl.BlockSpec((_BLK, hid), lambda i: (i, 0)),
        ],
        out_shape=[
            jax.ShapeDtypeStruct((n, hid), jnp.float32),
            jax.ShapeDtypeStruct((n, hid), jnp.float32),
        ],
    )(degp, x, w1, b1.reshape(1, hid))


def _layer_body(beta, degp_ref, part_ref, curs_ref, x0_ref, w_ref,
                cur_ref, curs_out_ref):
    dinv = _dinv_from_degp(degp_ref[...])
    s = part_ref[0] + part_ref[1] + curs_ref[...]
    agg = s * dinv[:, None]
    out = (1.0 - _ALPHA) * agg + _ALPHA * x0_ref[...]
    out = (1.0 - beta) * out + beta * jnp.dot(
        out, w_ref[...], preferred_element_type=jnp.float32)
    cur = jnp.maximum(out, 0.0)
    cur_ref[...] = cur
    curs_out_ref[...] = cur * dinv[:, None]


def _layer(beta, degp, part, curs, x0, w):
    n, hid = x0.shape
    grid = n // _BLK
    return pl.pallas_call(
        functools.partial(_layer_body, beta),
        grid=(grid,),
        in_specs=[
            pl.BlockSpec((_NC, _BLK, 128), lambda i: (0, i, 0)),
            pl.BlockSpec((_NC, _BLK, hid), lambda i: (0, i, 0)),
            pl.BlockSpec((_BLK, hid), lambda i: (i, 0)),
            pl.BlockSpec((_BLK, hid), lambda i: (i, 0)),
            pl.BlockSpec((hid, hid), lambda i: (0, 0)),
        ],
        out_specs=[
            pl.BlockSpec((_BLK, hid), lambda i: (i, 0)),
            pl.BlockSpec((_BLK, hid), lambda i: (i, 0)),
        ],
        out_shape=[
            jax.ShapeDtypeStruct((n, hid), jnp.float32),
            jax.ShapeDtypeStruct((n, hid), jnp.float32),
        ],
    )(degp, part, curs, x0, w)


def _final_body(cur_ref, w2_ref, b2_ref, o_ref):
    logits = jnp.dot(cur_ref[...], w2_ref[...],
                     preferred_element_type=jnp.float32) + b2_ref[...]
    m = jnp.max(logits, axis=1, keepdims=True)
    shifted = logits - m
    lse = jnp.log(jnp.sum(jnp.exp(shifted), axis=1, keepdims=True))
    o_ref[...] = shifted - lse


def _final(cur, w2, b2):
    n, hid = cur.shape
    nc = w2.shape[1]
    grid = n // _BLK
    return pl.pallas_call(
        _final_body,
        grid=(grid,),
        in_specs=[
            pl.BlockSpec((_BLK, hid), lambda i: (i, 0)),
            pl.BlockSpec((hid, nc), lambda i: (0, 0)),
            pl.BlockSpec((1, nc), lambda i: (0, 0)),
        ],
        out_specs=pl.BlockSpec((_BLK, nc), lambda i: (i, 0)),
        out_shape=jax.ShapeDtypeStruct((n, nc), jnp.float32),
    )(cur, w2, b2.reshape(1, nc))


def kernel(x, edge_index, y, W1, b1, Ws, W2, b2):
    n = x.shape[0]
    src = edge_index[0]
    dst = edge_index[1]
    degp = _sc_degree(dst, n)
    h, curs = _pre(degp, x, W1, b1)
    x0 = h
    cur = h
    for i in range(Ws.shape[0]):
        part = _sc_scatter(curs, src, dst)
        beta = float(np.log(_THETA / (i + 1) + 1.0))
        cur, curs = _layer(beta, degp, part, curs, x0, Ws[i])
    return _final(cur, W2, b2)


# speedup vs baseline: 1.9150x; 1.1207x over previous
import functools
import math

import jax
import jax.numpy as jnp
import numpy as np
from jax import lax
from jax.experimental import pallas as pl
from jax.experimental.pallas import tpu as pltpu
from jax.experimental.pallas import tpu_sc as plsc

_ALPHA = 0.1
_THETA = 0.5

_NC = 2
_NS = 16
_NW = _NC * _NS
_CHUNK = 128


def _sc_mesh():
    return plsc.VectorSubcoreMesh(
        core_axis_name="c", subcore_axis_name="s",
        num_cores=_NC, num_subcores=_NS)


def _pad_rows(n):
    per = -(-n // _NS)
    per = -(-per // 8) * 8
    return per * _NS, per


@functools.partial(jax.jit, static_argnums=(1,))
def _sc_degree(dst, n):
    e = dst.shape[0]
    ew = e // _NW
    nfull = ew // _CHUNK
    tail = ew % _CHUNK
    n_pad, rows_per_tile = _pad_rows(n)
    dcol = 128

    ones_rows = jnp.ones((_CHUNK, dcol), jnp.float32)
    zero_rows = jnp.zeros((_CHUNK, dcol), jnp.float32)

    @functools.partial(
        pl.kernel,
        out_type=jax.ShapeDtypeStruct((_NC, n_pad, dcol), jnp.float32),
        mesh=_sc_mesh(),
        scratch_types=[
            pltpu.VMEM((_CHUNK,), jnp.int32),
            pltpu.VMEM((tail,), jnp.int32),
            pltpu.VMEM((_CHUNK, dcol), jnp.float32),
            pltpu.VMEM((_CHUNK, dcol), jnp.float32),
            pltpu.VMEM_SHARED((n_pad, dcol), jnp.float32),
        ],
    )
    def k(dst_hbm, ones_hbm, zeros_hbm, out_hbm, dst_v, dstt_v, ones_v,
          zeros_v, acc_sh):
        cid = lax.axis_index("c")
        sid = lax.axis_index("s")
        wid = cid * _NS + sid
        base = wid * ew
        r0 = sid * rows_per_tile

        pltpu.sync_copy(ones_hbm, ones_v)
        pltpu.sync_copy(zeros_hbm, zeros_v)
        zc = _CHUNK
        for j in range((rows_per_tile + zc - 1) // zc):
            m = min(zc, rows_per_tile - j * zc)
            pltpu.sync_copy(zeros_v.at[pl.ds(0, m)],
                            acc_sh.at[pl.ds(r0 + j * zc, m)])
        plsc.subcore_barrier()

        @pl.loop(0, nfull)
        def body(i):
            eb = base + i * _CHUNK
            pltpu.sync_copy(dst_hbm.at[pl.ds(eb, _CHUNK)], dst_v)
            pltpu.sync_copy(ones_v, acc_sh.at[dst_v], add=True)

        if tail:
            pltpu.sync_copy(dst_hbm.at[pl.ds(base + nfull * _CHUNK, tail)],
                            dstt_v)
            pltpu.sync_copy(ones_v.at[pl.ds(0, tail)], acc_sh.at[dstt_v],
                            add=True)
        plsc.subcore_barrier()
        pltpu.sync_copy(acc_sh.at[pl.ds(r0, rows_per_tile)],
                        out_hbm.at[cid, pl.ds(r0, rows_per_tile)])

    return k(dst, ones_rows, zero_rows)


def _sc_scatter_build(n, d, e):
    ew = e // _NW
    nfull = ew // _CHUNK
    tail = ew % _CHUNK
    n_pad, rows_per_tile = _pad_rows(n)

    @functools.partial(
        pl.kernel,
        out_type=jax.ShapeDtypeStruct((_NC, n_pad, d), jnp.float32),
        mesh=_sc_mesh(),
        scratch_types=[
            pltpu.VMEM((_CHUNK,), jnp.int32),
            pltpu.VMEM((_CHUNK,), jnp.int32),
            pltpu.VMEM((tail,), jnp.int32),
            pltpu.VMEM((tail,), jnp.int32),
            pltpu.VMEM((_CHUNK, d), jnp.float32),
            pltpu.VMEM((tail, d), jnp.float32),
            pltpu.VMEM_SHARED((n_pad, d), jnp.float32),
            pltpu.SemaphoreType.DMA,
            pltpu.SemaphoreType.DMA,
            pltpu.SemaphoreType.DMA,
        ],
    )
    def k(table_hbm, src_hbm, dst_hbm, zeros_hbm, out_hbm,
          src_v, dst_v, srct_v, dstt_v, rows_v, rowst_v, acc_sh, sem,
          sem_is, sem_id):
        cid = lax.axis_index("c")
        sid = lax.axis_index("s")
        wid = cid * _NS + sid
        base = wid * ew
        r0 = sid * rows_per_tile

        pltpu.sync_copy(zeros_hbm, rows_v)
        zc = _CHUNK
        for j in range((rows_per_tile + zc - 1) // zc):
            m = min(zc, rows_per_tile - j * zc)
            pltpu.sync_copy(rows_v.at[pl.ds(0, m)],
                            acc_sh.at[pl.ds(r0 + j * zc, m)])
        plsc.subcore_barrier()

        @pl.loop(0, nfull)
        def body(i):
            eb = base + i * _CHUNK
            ca = pltpu.async_copy(src_hbm.at[pl.ds(eb, _CHUNK)], src_v,
                                  sem_is)
            cb = pltpu.async_copy(dst_hbm.at[pl.ds(eb, _CHUNK)], dst_v,
                                  sem_id)
            ca.wait()
            cb.wait()
            pltpu.async_copy(table_hbm.at[src_v], rows_v, sem).wait()
            pltpu.sync_copy(rows_v, acc_sh.at[dst_v], add=True)

        if tail:
            eb = base + nfull * _CHUNK
            pltpu.sync_copy(src_hbm.at[pl.ds(eb, tail)], srct_v)
            pltpu.sync_copy(dst_hbm.at[pl.ds(eb, tail)], dstt_v)
            pltpu.async_copy(table_hbm.at[srct_v], rowst_v, sem).wait()
            pltpu.sync_copy(rowst_v, acc_sh.at[dstt_v], add=True)
        plsc.subcore_barrier()
        pltpu.sync_copy(acc_sh.at[pl.ds(r0, rows_per_tile)],
                        out_hbm.at[cid, pl.ds(r0, rows_per_tile)])

    return k


@functools.partial(jax.jit, static_argnums=())
def _sc_scatter(table, src, dst):
    n, d = table.shape
    zero_rows = jnp.zeros((_CHUNK, d), jnp.float32)
    k = _sc_scatter_build(n, d, src.shape[0])
    return k(table, src, dst, zero_rows)


_BLK = 1000


def _dinv_from_degp(degp):
    deg = degp[0, :, 0] + degp[1, :, 0] + 1.0
    return lax.rsqrt(deg)


def _pre_body(degp_ref, x_ref, w1_ref, b1_ref, h_ref, hs_ref):
    dinv = _dinv_from_degp(degp_ref[...])
    h = jnp.maximum(
        jnp.dot(x_ref[...], w1_ref[...],
                preferred_element_type=jnp.float32) + b1_ref[...], 0.0)
    h_ref[...] = h
    hs_ref[...] = h * dinv[:, None]


def _pre(degp, x, w1, b1):
    n, dft = x.shape
    hid = w1.shape[1]
    grid = n // _BLK
    return pl.pallas_call(
        _pre_body,
        grid=(grid,),
        in_specs=[
            pl.BlockSpec((_NC, _BLK, 128), lambda i: (0, i, 0)),
            pl.BlockSpec((_BLK, dft), lambda i: (i, 0)),
            pl.BlockSpec((dft, hid), lambda i: (0, 0)),
            pl.BlockSpec((1, hid), lambda i: (0, 0)),
        ],
        out_specs=[
            pl.BlockSpec((_BLK, hid), lambda i: (i, 0)),
            pl.BlockSpec((_BLK, hid), lambda i: (i, 0)),
        ],
        out_shape=[
            jax.ShapeDtypeStruct((n, hid), jnp.float32),
            jax.ShapeDtypeStruct((n, hid), jnp.float32),
        ],
    )(degp, x, w1, b1.reshape(1, hid))


def _layer_body(beta, degp_ref, part_ref, curs_ref, x0_ref, w_ref,
                cur_ref, curs_out_ref):
    dinv = _dinv_from_degp(degp_ref[...])
    s = part_ref[0] + part_ref[1] + curs_ref[...]
    agg = s * dinv[:, None]
    out = (1.0 - _ALPHA) * agg + _ALPHA * x0_ref[...]
    out = (1.0 - beta) * out + beta * jnp.dot(
        out, w_ref[...], preferred_element_type=jnp.float32)
    cur = jnp.maximum(out, 0.0)
    cur_ref[...] = cur
    curs_out_ref[...] = cur * dinv[:, None]


def _layer(beta, degp, part, curs, x0, w):
    n, hid = x0.shape
    grid = n // _BLK
    return pl.pallas_call(
        functools.partial(_layer_body, beta),
        grid=(grid,),
        in_specs=[
            pl.BlockSpec((_NC, _BLK, 128), lambda i: (0, i, 0)),
            pl.BlockSpec((_NC, _BLK, hid), lambda i: (0, i, 0)),
            pl.BlockSpec((_BLK, hid), lambda i: (i, 0)),
            pl.BlockSpec((_BLK, hid), lambda i: (i, 0)),
            pl.BlockSpec((hid, hid), lambda i: (0, 0)),
        ],
        out_specs=[
            pl.BlockSpec((_BLK, hid), lambda i: (i, 0)),
            pl.BlockSpec((_BLK, hid), lambda i: (i, 0)),
        ],
        out_shape=[
            jax.ShapeDtypeStruct((n, hid), jnp.float32),
            jax.ShapeDtypeStruct((n, hid), jnp.float32),
        ],
    )(degp, part, curs, x0, w)


def _final_body(cur_ref, w2_ref, b2_ref, o_ref):
    logits = jnp.dot(cur_ref[...], w2_ref[...],
                     preferred_element_type=jnp.float32) + b2_ref[...]
    m = jnp.max(logits, axis=1, keepdims=True)
    shifted = logits - m
    lse = jnp.log(jnp.sum(jnp.exp(shifted), axis=1, keepdims=True))
    o_ref[...] = shifted - lse


def _final(cur, w2, b2):
    n, hid = cur.shape
    nc = w2.shape[1]
    grid = n // _BLK
    return pl.pallas_call(
        _final_body,
        grid=(grid,),
        in_specs=[
            pl.BlockSpec((_BLK, hid), lambda i: (i, 0)),
            pl.BlockSpec((hid, nc), lambda i: (0, 0)),
            pl.BlockSpec((1, nc), lambda i: (0, 0)),
        ],
        out_specs=pl.BlockSpec((_BLK, nc), lambda i: (i, 0)),
        out_shape=jax.ShapeDtypeStruct((n, nc), jnp.float32),
    )(cur, w2, b2.reshape(1, nc))


def kernel(x, edge_index, y, W1, b1, Ws, W2, b2):
    n = x.shape[0]
    src = edge_index[0]
    dst = edge_index[1]
    degp = _sc_degree(dst, n)
    h, curs = _pre(degp, x, W1, b1)
    x0 = h
    cur = h
    for i in range(Ws.shape[0]):
        part = _sc_scatter(curs, src, dst)
        beta = float(np.log(_THETA / (i + 1) + 1.0))
        cur, curs = _layer(beta, degp, part, curs, x0, Ws[i])
    return _final(cur, W2, b2)


# speedup vs baseline: 2.2103x; 1.1542x over previous
import functools
import math

import jax
import jax.numpy as jnp
import numpy as np
from jax import lax
from jax.experimental import pallas as pl
from jax.experimental.pallas import tpu as pltpu
from jax.experimental.pallas import tpu_sc as plsc

_ALPHA = 0.1
_THETA = 0.5

_NC = 2
_NS = 16
_NW = _NC * _NS
_CHUNK = 128


def _sc_mesh():
    return plsc.VectorSubcoreMesh(
        core_axis_name="c", subcore_axis_name="s",
        num_cores=_NC, num_subcores=_NS)


def _pad_rows(n):
    per = -(-n // _NS)
    per = -(-per // 8) * 8
    return per * _NS, per


@functools.partial(jax.jit, static_argnums=(1,))
def _sc_degree(dst, n):
    e = dst.shape[0]
    ew = e // _NW
    nfull = ew // _CHUNK
    tail = ew % _CHUNK
    n_pad, rows_per_tile = _pad_rows(n)
    dcol = 128

    ones_rows = jnp.ones((_CHUNK, dcol), jnp.float32)
    zero_rows = jnp.zeros((_CHUNK, dcol), jnp.float32)

    @functools.partial(
        pl.kernel,
        out_type=jax.ShapeDtypeStruct((_NC, n_pad, dcol), jnp.float32),
        mesh=_sc_mesh(),
        scratch_types=[
            pltpu.VMEM((_CHUNK,), jnp.int32),
            pltpu.VMEM((tail,), jnp.int32),
            pltpu.VMEM((_CHUNK, dcol), jnp.float32),
            pltpu.VMEM((_CHUNK, dcol), jnp.float32),
            pltpu.VMEM_SHARED((n_pad, dcol), jnp.float32),
        ],
    )
    def k(dst_hbm, ones_hbm, zeros_hbm, out_hbm, dst_v, dstt_v, ones_v,
          zeros_v, acc_sh):
        cid = lax.axis_index("c")
        sid = lax.axis_index("s")
        wid = cid * _NS + sid
        base = wid * ew
        r0 = sid * rows_per_tile

        pltpu.sync_copy(ones_hbm, ones_v)
        pltpu.sync_copy(zeros_hbm, zeros_v)
        zc = _CHUNK
        for j in range((rows_per_tile + zc - 1) // zc):
            m = min(zc, rows_per_tile - j * zc)
            pltpu.sync_copy(zeros_v.at[pl.ds(0, m)],
                            acc_sh.at[pl.ds(r0 + j * zc, m)])
        plsc.subcore_barrier()

        @pl.loop(0, nfull)
        def body(i):
            eb = base + i * _CHUNK
            pltpu.sync_copy(dst_hbm.at[pl.ds(eb, _CHUNK)], dst_v)
            pltpu.sync_copy(ones_v, acc_sh.at[dst_v], add=True)

        if tail:
            pltpu.sync_copy(dst_hbm.at[pl.ds(base + nfull * _CHUNK, tail)],
                            dstt_v)
            pltpu.sync_copy(ones_v.at[pl.ds(0, tail)], acc_sh.at[dstt_v],
                            add=True)
        plsc.subcore_barrier()
        pltpu.sync_copy(acc_sh.at[pl.ds(r0, rows_per_tile)],
                        out_hbm.at[cid, pl.ds(r0, rows_per_tile)])

    return k(dst, ones_rows, zero_rows)


def _sc_scatter_build(n, d, e):
    ew = e // _NW
    nfull = ew // _CHUNK
    tail = ew % _CHUNK
    n_pad, rows_per_tile = _pad_rows(n)

    @functools.partial(
        pl.kernel,
        out_type=jax.ShapeDtypeStruct((_NC, n_pad, d), jnp.float32),
        mesh=_sc_mesh(),
        scratch_types=[
            pltpu.VMEM((_CHUNK,), jnp.int32),
            pltpu.VMEM((_CHUNK,), jnp.int32),
            pltpu.VMEM((_CHUNK,), jnp.int32),
            pltpu.VMEM((_CHUNK,), jnp.int32),
            pltpu.VMEM((tail,), jnp.int32),
            pltpu.VMEM((tail,), jnp.int32),
            pltpu.VMEM((_CHUNK, d), jnp.float32),
            pltpu.VMEM((tail, d), jnp.float32),
            pltpu.VMEM_SHARED((n_pad, d), jnp.float32),
            pltpu.SemaphoreType.DMA,
            pltpu.SemaphoreType.DMA,
            pltpu.SemaphoreType.DMA,
            pltpu.SemaphoreType.DMA,
            pltpu.SemaphoreType.DMA,
        ],
    )
    def k(table_hbm, src_hbm, dst_hbm, zeros_hbm, out_hbm,
          src_a, dst_a, src_b, dst_b, srct_v, dstt_v, rows_v, rowst_v,
          acc_sh, sem, sas, sad, sbs, sbd):
        cid = lax.axis_index("c")
        sid = lax.axis_index("s")
        wid = cid * _NS + sid
        base = wid * ew
        r0 = sid * rows_per_tile

        def issue(sv, dv, ss, sd, ci):
            eb = base + ci * _CHUNK
            ca = pltpu.async_copy(src_hbm.at[pl.ds(eb, _CHUNK)], sv, ss)
            cb = pltpu.async_copy(dst_hbm.at[pl.ds(eb, _CHUNK)], dv, sd)
            return ca, cb

        def drain(sv, dv, ss, sd, ci):
            eb = base + ci * _CHUNK
            pltpu.make_async_copy(src_hbm.at[pl.ds(eb, _CHUNK)], sv,
                                  ss).wait()
            pltpu.make_async_copy(dst_hbm.at[pl.ds(eb, _CHUNK)], dv,
                                  sd).wait()

        def gat_scat(sv, dv):
            pltpu.async_copy(table_hbm.at[sv], rows_v, sem).wait()
            pltpu.sync_copy(rows_v, acc_sh.at[dv], add=True)

        pltpu.sync_copy(zeros_hbm, rows_v)
        zc = _CHUNK
        for j in range((rows_per_tile + zc - 1) // zc):
            m = min(zc, rows_per_tile - j * zc)
            pltpu.sync_copy(rows_v.at[pl.ds(0, m)],
                            acc_sh.at[pl.ds(r0 + j * zc, m)])
        plsc.subcore_barrier()

        issue(src_a, dst_a, sas, sad, 0)

        @pl.loop(0, nfull // 2 - 1)
        def body(gi):
            c0 = 2 * gi
            drain(src_a, dst_a, sas, sad, c0)
            issue(src_b, dst_b, sbs, sbd, c0 + 1)
            gat_scat(src_a, dst_a)
            drain(src_b, dst_b, sbs, sbd, c0 + 1)
            issue(src_a, dst_a, sas, sad, c0 + 2)
            gat_scat(src_b, dst_b)

        c0 = nfull - 2
        drain(src_a, dst_a, sas, sad, c0)
        issue(src_b, dst_b, sbs, sbd, c0 + 1)
        gat_scat(src_a, dst_a)
        drain(src_b, dst_b, sbs, sbd, c0 + 1)
        gat_scat(src_b, dst_b)

        if tail:
            eb = base + nfull * _CHUNK
            pltpu.sync_copy(src_hbm.at[pl.ds(eb, tail)], srct_v)
            pltpu.sync_copy(dst_hbm.at[pl.ds(eb, tail)], dstt_v)
            pltpu.async_copy(table_hbm.at[srct_v], rowst_v, sem).wait()
            pltpu.sync_copy(rowst_v, acc_sh.at[dstt_v], add=True)
        plsc.subcore_barrier()
        pltpu.sync_copy(acc_sh.at[pl.ds(r0, rows_per_tile)],
                        out_hbm.at[cid, pl.ds(r0, rows_per_tile)])

    return k


@functools.partial(jax.jit, static_argnums=())
def _sc_scatter(table, src, dst):
    n, d = table.shape
    zero_rows = jnp.zeros((_CHUNK, d), jnp.float32)
    k = _sc_scatter_build(n, d, src.shape[0])
    return k(table, src, dst, zero_rows)


_BLK = 1000


def _dinv_from_degp(degp):
    deg = degp[0, :, 0] + degp[1, :, 0] + 1.0
    return lax.rsqrt(deg)


def _pre_body(degp_ref, x_ref, w1_ref, b1_ref, h_ref, hs_ref):
    dinv = _dinv_from_degp(degp_ref[...])
    h = jnp.maximum(
        jnp.dot(x_ref[...], w1_ref[...],
                preferred_element_type=jnp.float32) + b1_ref[...], 0.0)
    h_ref[...] = h
    hs_ref[...] = h * dinv[:, None]


def _pre(degp, x, w1, b1):
    n, dft = x.shape
    hid = w1.shape[1]
    grid = n // _BLK
    return pl.pallas_call(
        _pre_body,
        grid=(grid,),
        in_specs=[
            pl.BlockSpec((_NC, _BLK, 128), lambda i: (0, i, 0)),
            pl.BlockSpec((_BLK, dft), lambda i: (i, 0)),
            pl.BlockSpec((dft, hid), lambda i: (0, 0)),
            pl.BlockSpec((1, hid), lambda i: (0, 0)),
        ],
        out_specs=[
            pl.BlockSpec((_BLK, hid), lambda i: (i, 0)),
            pl.BlockSpec((_BLK, hid), lambda i: (i, 0)),
        ],
        out_shape=[
            jax.ShapeDtypeStruct((n, hid), jnp.float32),
            jax.ShapeDtypeStruct((n, hid), jnp.float32),
        ],
    )(degp, x, w1, b1.reshape(1, hid))


def _layer_body(beta, degp_ref, part_ref, curs_ref, x0_ref, w_ref,
                cur_ref, curs_out_ref):
    dinv = _dinv_from_degp(degp_ref[...])
    s = part_ref[0] + part_ref[1] + curs_ref[...]
    agg = s * dinv[:, None]
    out = (1.0 - _ALPHA) * agg + _ALPHA * x0_ref[...]
    out = (1.0 - beta) * out + beta * jnp.dot(
        out, w_ref[...], preferred_element_type=jnp.float32)
    cur = jnp.maximum(out, 0.0)
    cur_ref[...] = cur
    curs_out_ref[...] = cur * dinv[:, None]


def _layer(beta, degp, part, curs, x0, w):
    n, hid = x0.shape
    grid = n // _BLK
    return pl.pallas_call(
        functools.partial(_layer_body, beta),
        grid=(grid,),
        in_specs=[
            pl.BlockSpec((_NC, _BLK, 128), lambda i: (0, i, 0)),
            pl.BlockSpec((_NC, _BLK, hid), lambda i: (0, i, 0)),
            pl.BlockSpec((_BLK, hid), lambda i: (i, 0)),
            pl.BlockSpec((_BLK, hid), lambda i: (i, 0)),
            pl.BlockSpec((hid, hid), lambda i: (0, 0)),
        ],
        out_specs=[
            pl.BlockSpec((_BLK, hid), lambda i: (i, 0)),
            pl.BlockSpec((_BLK, hid), lambda i: (i, 0)),
        ],
        out_shape=[
            jax.ShapeDtypeStruct((n, hid), jnp.float32),
            jax.ShapeDtypeStruct((n, hid), jnp.float32),
        ],
    )(degp, part, curs, x0, w)


def _final_body(cur_ref, w2_ref, b2_ref, o_ref):
    logits = jnp.dot(cur_ref[...], w2_ref[...],
                     preferred_element_type=jnp.float32) + b2_ref[...]
    m = jnp.max(logits, axis=1, keepdims=True)
    shifted = logits - m
    lse = jnp.log(jnp.sum(jnp.exp(shifted), axis=1, keepdims=True))
    o_ref[...] = shifted - lse


def _final(cur, w2, b2):
    n, hid = cur.shape
    nc = w2.shape[1]
    grid = n // _BLK
    return pl.pallas_call(
        _final_body,
        grid=(grid,),
        in_specs=[
            pl.BlockSpec((_BLK, hid), lambda i: (i, 0)),
            pl.BlockSpec((hid, nc), lambda i: (0, 0)),
            pl.BlockSpec((1, nc), lambda i: (0, 0)),
        ],
        out_specs=pl.BlockSpec((_BLK, nc), lambda i: (i, 0)),
        out_shape=jax.ShapeDtypeStruct((n, nc), jnp.float32),
    )(cur, w2, b2.reshape(1, nc))


def kernel(x, edge_index, y, W1, b1, Ws, W2, b2):
    n = x.shape[0]
    src = edge_index[0]
    dst = edge_index[1]
    degp = _sc_degree(dst, n)
    h, curs = _pre(degp, x, W1, b1)
    x0 = h
    cur = h
    for i in range(Ws.shape[0]):
        part = _sc_scatter(curs, src, dst)
        beta = float(np.log(_THETA / (i + 1) + 1.0))
        cur, curs = _layer(beta, degp, part, curs, x0, Ws[i])
    return _final(cur, W2, b2)


# speedup vs baseline: 2.2813x; 1.0321x over previous
import functools
import math

import jax
import jax.numpy as jnp
import numpy as np
from jax import lax
from jax.experimental import pallas as pl
from jax.experimental.pallas import tpu as pltpu
from jax.experimental.pallas import tpu_sc as plsc

_ALPHA = 0.1
_THETA = 0.5

_NC = 2
_NS = 16
_NW = _NC * _NS
_CHUNK = 128


def _sc_mesh():
    return plsc.VectorSubcoreMesh(
        core_axis_name="c", subcore_axis_name="s",
        num_cores=_NC, num_subcores=_NS)


def _pad_rows(n):
    per = -(-n // _NS)
    per = -(-per // 8) * 8
    return per * _NS, per


@functools.partial(jax.jit, static_argnums=(1,))
def _sc_degree(dst, n):
    e = dst.shape[0]
    ew = e // _NW
    nfull = ew // _CHUNK
    tail = ew % _CHUNK
    n_pad, rows_per_tile = _pad_rows(n)
    dcol = 128

    ones_rows = jnp.ones((_CHUNK, dcol), jnp.float32)
    zero_rows = jnp.zeros((_CHUNK, dcol), jnp.float32)

    @functools.partial(
        pl.kernel,
        out_type=jax.ShapeDtypeStruct((_NC, n_pad, dcol), jnp.float32),
        mesh=_sc_mesh(),
        scratch_types=[
            pltpu.VMEM((_CHUNK,), jnp.int32),
            pltpu.VMEM((_CHUNK,), jnp.int32),
            pltpu.VMEM((tail,), jnp.int32),
            pltpu.VMEM((_CHUNK, dcol), jnp.float32),
            pltpu.VMEM((_CHUNK, dcol), jnp.float32),
            pltpu.VMEM_SHARED((n_pad, dcol), jnp.float32),
            pltpu.SemaphoreType.DMA,
            pltpu.SemaphoreType.DMA,
        ],
    )
    def k(dst_hbm, ones_hbm, zeros_hbm, out_hbm, dst_a, dst_b, dstt_v,
          ones_v, zeros_v, acc_sh, sa, sb):
        cid = lax.axis_index("c")
        sid = lax.axis_index("s")
        wid = cid * _NS + sid
        base = wid * ew
        r0 = sid * rows_per_tile

        def issue(dv, ss, ci):
            pltpu.async_copy(
                dst_hbm.at[pl.ds(base + ci * _CHUNK, _CHUNK)], dv, ss)

        def drain(dv, ss, ci):
            pltpu.make_async_copy(
                dst_hbm.at[pl.ds(base + ci * _CHUNK, _CHUNK)], dv,
                ss).wait()

        pltpu.sync_copy(ones_hbm, ones_v)
        pltpu.sync_copy(zeros_hbm, zeros_v)
        zc = _CHUNK
        for j in range((rows_per_tile + zc - 1) // zc):
            m = min(zc, rows_per_tile - j * zc)
            pltpu.sync_copy(zeros_v.at[pl.ds(0, m)],
                            acc_sh.at[pl.ds(r0 + j * zc, m)])
        plsc.subcore_barrier()

        issue(dst_a, sa, 0)

        @pl.loop(0, nfull // 2 - 1)
        def body(gi):
            c0 = 2 * gi
            drain(dst_a, sa, c0)
            issue(dst_b, sb, c0 + 1)
            pltpu.sync_copy(ones_v, acc_sh.at[dst_a], add=True)
            drain(dst_b, sb, c0 + 1)
            issue(dst_a, sa, c0 + 2)
            pltpu.sync_copy(ones_v, acc_sh.at[dst_b], add=True)

        c0 = nfull - 2
        drain(dst_a, sa, c0)
        issue(dst_b, sb, c0 + 1)
        pltpu.sync_copy(ones_v, acc_sh.at[dst_a], add=True)
        drain(dst_b, sb, c0 + 1)
        pltpu.sync_copy(ones_v, acc_sh.at[dst_b], add=True)

        if tail:
            pltpu.sync_copy(dst_hbm.at[pl.ds(base + nfull * _CHUNK, tail)],
                            dstt_v)
            pltpu.sync_copy(ones_v.at[pl.ds(0, tail)], acc_sh.at[dstt_v],
                            add=True)
        plsc.subcore_barrier()
        pltpu.sync_copy(acc_sh.at[pl.ds(r0, rows_per_tile)],
                        out_hbm.at[cid, pl.ds(r0, rows_per_tile)])

    return k(dst, ones_rows, zero_rows)


def _sc_scatter_build(n, d, e):
    ew = e // _NW
    nfull = ew // _CHUNK
    tail = ew % _CHUNK
    n_pad, rows_per_tile = _pad_rows(n)

    @functools.partial(
        pl.kernel,
        out_type=jax.ShapeDtypeStruct((_NC, n_pad, d), jnp.float32),
        mesh=_sc_mesh(),
        scratch_types=[
            pltpu.VMEM((_CHUNK,), jnp.int32),
            pltpu.VMEM((_CHUNK,), jnp.int32),
            pltpu.VMEM((_CHUNK,), jnp.int32),
            pltpu.VMEM((_CHUNK,), jnp.int32),
            pltpu.VMEM((tail,), jnp.int32),
            pltpu.VMEM((tail,), jnp.int32),
            pltpu.VMEM((_CHUNK, d), jnp.float32),
            pltpu.VMEM((tail, d), jnp.float32),
            pltpu.VMEM_SHARED((n_pad, d), jnp.float32),
            pltpu.SemaphoreType.DMA,
            pltpu.SemaphoreType.DMA,
            pltpu.SemaphoreType.DMA,
            pltpu.SemaphoreType.DMA,
            pltpu.SemaphoreType.DMA,
        ],
    )
    def k(table_hbm, src_hbm, dst_hbm, zeros_hbm, out_hbm,
          src_a, dst_a, src_b, dst_b, srct_v, dstt_v, rows_v, rowst_v,
          acc_sh, sem, sas, sad, sbs, sbd):
        cid = lax.axis_index("c")
        sid = lax.axis_index("s")
        wid = cid * _NS + sid
        base = wid * ew
        r0 = sid * rows_per_tile

        def issue(sv, dv, ss, sd, ci):
            eb = base + ci * _CHUNK
            ca = pltpu.async_copy(src_hbm.at[pl.ds(eb, _CHUNK)], sv, ss)
            cb = pltpu.async_copy(dst_hbm.at[pl.ds(eb, _CHUNK)], dv, sd)
            return ca, cb

        def drain(sv, dv, ss, sd, ci):
            eb = base + ci * _CHUNK
            pltpu.make_async_copy(src_hbm.at[pl.ds(eb, _CHUNK)], sv,
                                  ss).wait()
            pltpu.make_async_copy(dst_hbm.at[pl.ds(eb, _CHUNK)], dv,
                                  sd).wait()

        def gat_scat(sv, dv):
            pltpu.async_copy(table_hbm.at[sv], rows_v, sem).wait()
            pltpu.sync_copy(rows_v, acc_sh.at[dv], add=True)

        pltpu.sync_copy(zeros_hbm, rows_v)
        zc = _CHUNK
        for j in range((rows_per_tile + zc - 1) // zc):
            m = min(zc, rows_per_tile - j * zc)
            pltpu.sync_copy(rows_v.at[pl.ds(0, m)],
                            acc_sh.at[pl.ds(r0 + j * zc, m)])
        plsc.subcore_barrier()

        issue(src_a, dst_a, sas, sad, 0)

        @pl.loop(0, nfull // 2 - 1)
        def body(gi):
            c0 = 2 * gi
            drain(src_a, dst_a, sas, sad, c0)
            issue(src_b, dst_b, sbs, sbd, c0 + 1)
            gat_scat(src_a, dst_a)
            drain(src_b, dst_b, sbs, sbd, c0 + 1)
            issue(src_a, dst_a, sas, sad, c0 + 2)
            gat_scat(src_b, dst_b)

        c0 = nfull - 2
        drain(src_a, dst_a, sas, sad, c0)
        issue(src_b, dst_b, sbs, sbd, c0 + 1)
        gat_scat(src_a, dst_a)
        drain(src_b, dst_b, sbs, sbd, c0 + 1)
        gat_scat(src_b, dst_b)

        if tail:
            eb = base + nfull * _CHUNK
            pltpu.sync_copy(src_hbm.at[pl.ds(eb, tail)], srct_v)
            pltpu.sync_copy(dst_hbm.at[pl.ds(eb, tail)], dstt_v)
            pltpu.async_copy(table_hbm.at[srct_v], rowst_v, sem).wait()
            pltpu.sync_copy(rowst_v, acc_sh.at[dstt_v], add=True)
        plsc.subcore_barrier()
        pltpu.sync_copy(acc_sh.at[pl.ds(r0, rows_per_tile)],
                        out_hbm.at[cid, pl.ds(r0, rows_per_tile)])

    return k


@functools.partial(jax.jit, static_argnums=())
def _sc_scatter(table, src, dst):
    n, d = table.shape
    zero_rows = jnp.zeros((_CHUNK, d), jnp.float32)
    k = _sc_scatter_build(n, d, src.shape[0])
    return k(table, src, dst, zero_rows)


_BLK = 1000


def _dinv_from_degp(degp):
    deg = degp[0, :, 0] + degp[1, :, 0] + 1.0
    return lax.rsqrt(deg)


def _pre_body(degp_ref, x_ref, w1_ref, b1_ref, h_ref, hs_ref):
    dinv = _dinv_from_degp(degp_ref[...])
    h = jnp.maximum(
        jnp.dot(x_ref[...], w1_ref[...],
                preferred_element_type=jnp.float32) + b1_ref[...], 0.0)
    h_ref[...] = h
    hs_ref[...] = h * dinv[:, None]


def _pre(degp, x, w1, b1):
    n, dft = x.shape
    hid = w1.shape[1]
    grid = n // _BLK
    return pl.pallas_call(
        _pre_body,
        grid=(grid,),
        in_specs=[
            pl.BlockSpec((_NC, _BLK, 128), lambda i: (0, i, 0)),
            pl.BlockSpec((_BLK, dft), lambda i: (i, 0)),
            pl.BlockSpec((dft, hid), lambda i: (0, 0)),
            pl.BlockSpec((1, hid), lambda i: (0, 0)),
        ],
        out_specs=[
            pl.BlockSpec((_BLK, hid), lambda i: (i, 0)),
            pl.BlockSpec((_BLK, hid), lambda i: (i, 0)),
        ],
        out_shape=[
            jax.ShapeDtypeStruct((n, hid), jnp.float32),
            jax.ShapeDtypeStruct((n, hid), jnp.float32),
        ],
    )(degp, x, w1, b1.reshape(1, hid))


def _layer_body(beta, degp_ref, part_ref, curs_ref, x0_ref, w_ref,
                cur_ref, curs_out_ref):
    dinv = _dinv_from_degp(degp_ref[...])
    s = part_ref[0] + part_ref[1] + curs_ref[...]
    agg = s * dinv[:, None]
    out = (1.0 - _ALPHA) * agg + _ALPHA * x0_ref[...]
    out = (1.0 - beta) * out + beta * jnp.dot(
        out, w_ref[...], preferred_element_type=jnp.float32)
    cur = jnp.maximum(out, 0.0)
    cur_ref[...] = cur
    curs_out_ref[...] = cur * dinv[:, None]


def _layer(beta, degp, part, curs, x0, w):
    n, hid = x0.shape
    grid = n // _BLK
    return pl.pallas_call(
        functools.partial(_layer_body, beta),
        grid=(grid,),
        in_specs=[
            pl.BlockSpec((_NC, _BLK, 128), lambda i: (0, i, 0)),
            pl.BlockSpec((_NC, _BLK, hid), lambda i: (0, i, 0)),
            pl.BlockSpec((_BLK, hid), lambda i: (i, 0)),
            pl.BlockSpec((_BLK, hid), lambda i: (i, 0)),
            pl.BlockSpec((hid, hid), lambda i: (0, 0)),
        ],
        out_specs=[
            pl.BlockSpec((_BLK, hid), lambda i: (i, 0)),
            pl.BlockSpec((_BLK, hid), lambda i: (i, 0)),
        ],
        out_shape=[
            jax.ShapeDtypeStruct((n, hid), jnp.float32),
            jax.ShapeDtypeStruct((n, hid), jnp.float32),
        ],
    )(degp, part, curs, x0, w)


def _final_body(cur_ref, w2_ref, b2_ref, o_ref):
    logits = jnp.dot(cur_ref[...], w2_ref[...],
                     preferred_element_type=jnp.float32) + b2_ref[...]
    m = jnp.max(logits, axis=1, keepdims=True)
    shifted = logits - m
    lse = jnp.log(jnp.sum(jnp.exp(shifted), axis=1, keepdims=True))
    o_ref[...] = shifted - lse


def _final(cur, w2, b2):
    n, hid = cur.shape
    nc = w2.shape[1]
    grid = n // _BLK
    return pl.pallas_call(
        _final_body,
        grid=(grid,),
        in_specs=[
            pl.BlockSpec((_BLK, hid), lambda i: (i, 0)),
            pl.BlockSpec((hid, nc), lambda i: (0, 0)),
            pl.BlockSpec((1, nc), lambda i: (0, 0)),
        ],
        out_specs=pl.BlockSpec((_BLK, nc), lambda i: (i, 0)),
        out_shape=jax.ShapeDtypeStruct((n, nc), jnp.float32),
    )(cur, w2, b2.reshape(1, nc))


def kernel(x, edge_index, y, W1, b1, Ws, W2, b2):
    n = x.shape[0]
    src = edge_index[0]
    dst = edge_index[1]
    degp = _sc_degree(dst, n)
    h, curs = _pre(degp, x, W1, b1)
    x0 = h
    cur = h
    for i in range(Ws.shape[0]):
        part = _sc_scatter(curs, src, dst)
        beta = float(np.log(_THETA / (i + 1) + 1.0))
        cur, curs = _layer(beta, degp, part, curs, x0, Ws[i])
    return _final(cur, W2, b2)


# speedup vs baseline: 2.5647x; 1.1242x over previous
import functools
import math

import jax
import jax.numpy as jnp
import numpy as np
from jax import lax
from jax.experimental import pallas as pl
from jax.experimental.pallas import tpu as pltpu
from jax.experimental.pallas import tpu_sc as plsc

_ALPHA = 0.1
_THETA = 0.5

_NC = 2
_NS = 16
_NW = _NC * _NS
_CHUNK = 128


def _sc_mesh():
    return plsc.VectorSubcoreMesh(
        core_axis_name="c", subcore_axis_name="s",
        num_cores=_NC, num_subcores=_NS)


def _pad_rows(n):
    per = -(-n // _NS)
    per = -(-per // 8) * 8
    return per * _NS, per


@functools.partial(jax.jit, static_argnums=(1,))
def _sc_degree(dst, n):
    e = dst.shape[0]
    ew = e // _NW
    nfull = ew // _CHUNK
    tail = ew % _CHUNK
    n_pad, rows_per_tile = _pad_rows(n)
    dcol = 128

    ones_rows = jnp.ones((_CHUNK, dcol), jnp.float32)
    zero_rows = jnp.zeros((_CHUNK, dcol), jnp.float32)

    @functools.partial(
        pl.kernel,
        out_type=jax.ShapeDtypeStruct((_NC, n_pad, dcol), jnp.float32),
        mesh=_sc_mesh(),
        scratch_types=[
            pltpu.VMEM((_CHUNK,), jnp.int32),
            pltpu.VMEM((_CHUNK,), jnp.int32),
            pltpu.VMEM((tail,), jnp.int32),
            pltpu.VMEM((_CHUNK, dcol), jnp.float32),
            pltpu.VMEM((_CHUNK, dcol), jnp.float32),
            pltpu.VMEM_SHARED((n_pad, dcol), jnp.float32),
            pltpu.SemaphoreType.DMA,
            pltpu.SemaphoreType.DMA,
        ],
    )
    def k(dst_hbm, ones_hbm, zeros_hbm, out_hbm, dst_a, dst_b, dstt_v,
          ones_v, zeros_v, acc_sh, sa, sb):
        cid = lax.axis_index("c")
        sid = lax.axis_index("s")
        wid = cid * _NS + sid
        base = wid * ew
        r0 = sid * rows_per_tile

        def issue(dv, ss, ci):
            pltpu.async_copy(
                dst_hbm.at[pl.ds(base + ci * _CHUNK, _CHUNK)], dv, ss)

        def drain(dv, ss, ci):
            pltpu.make_async_copy(
                dst_hbm.at[pl.ds(base + ci * _CHUNK, _CHUNK)], dv,
                ss).wait()

        pltpu.sync_copy(ones_hbm, ones_v)
        pltpu.sync_copy(zeros_hbm, zeros_v)
        zc = _CHUNK
        for j in range((rows_per_tile + zc - 1) // zc):
            m = min(zc, rows_per_tile - j * zc)
            pltpu.sync_copy(zeros_v.at[pl.ds(0, m)],
                            acc_sh.at[pl.ds(r0 + j * zc, m)])
        plsc.subcore_barrier()

        issue(dst_a, sa, 0)

        @pl.loop(0, nfull // 2 - 1)
        def body(gi):
            c0 = 2 * gi
            drain(dst_a, sa, c0)
            issue(dst_b, sb, c0 + 1)
            pltpu.sync_copy(ones_v, acc_sh.at[dst_a], add=True)
            drain(dst_b, sb, c0 + 1)
            issue(dst_a, sa, c0 + 2)
            pltpu.sync_copy(ones_v, acc_sh.at[dst_b], add=True)

        c0 = nfull - 2
        drain(dst_a, sa, c0)
        issue(dst_b, sb, c0 + 1)
        pltpu.sync_copy(ones_v, acc_sh.at[dst_a], add=True)
        drain(dst_b, sb, c0 + 1)
        pltpu.sync_copy(ones_v, acc_sh.at[dst_b], add=True)

        if tail:
            pltpu.sync_copy(dst_hbm.at[pl.ds(base + nfull * _CHUNK, tail)],
                            dstt_v)
            pltpu.sync_copy(ones_v.at[pl.ds(0, tail)], acc_sh.at[dstt_v],
                            add=True)
        plsc.subcore_barrier()
        pltpu.sync_copy(acc_sh.at[pl.ds(r0, rows_per_tile)],
                        out_hbm.at[cid, pl.ds(r0, rows_per_tile)])

    return k(dst, ones_rows, zero_rows)


def _sc_scatter_build(n, d, e):
    ew = e // _NW
    nfull = ew // _CHUNK
    tail = ew % _CHUNK
    n_pad, rows_per_tile = _pad_rows(n)

    @functools.partial(
        pl.kernel,
        out_type=jax.ShapeDtypeStruct((_NC, n_pad, d), jnp.float32),
        mesh=_sc_mesh(),
        scratch_types=[
            pltpu.VMEM((_CHUNK,), jnp.int32),
            pltpu.VMEM((_CHUNK,), jnp.int32),
            pltpu.VMEM((_CHUNK,), jnp.int32),
            pltpu.VMEM((_CHUNK,), jnp.int32),
            pltpu.VMEM((tail,), jnp.int32),
            pltpu.VMEM((tail,), jnp.int32),
            pltpu.VMEM((_CHUNK, d), jnp.float32),
            pltpu.VMEM((_CHUNK, d), jnp.float32),
            pltpu.VMEM((tail, d), jnp.float32),
            pltpu.VMEM_SHARED((n_pad, d), jnp.float32),
        ] + [pltpu.SemaphoreType.DMA] * 8,
    )
    def k(table_hbm, src_hbm, dst_hbm, zeros_hbm, out_hbm,
          src_a, dst_a, src_b, dst_b, srct_v, dstt_v, rows_a, rows_b,
          rowst_v, acc_sh, sga, sgb, ssa, ssb, sas, sad, sbs, sbd):
        cid = lax.axis_index("c")
        sid = lax.axis_index("s")
        wid = cid * _NS + sid
        base = wid * ew
        r0 = sid * rows_per_tile

        def issue(sv, dv, ss, sd, ci):
            eb = base + ci * _CHUNK
            ca = pltpu.async_copy(src_hbm.at[pl.ds(eb, _CHUNK)], sv, ss)
            cb = pltpu.async_copy(dst_hbm.at[pl.ds(eb, _CHUNK)], dv, sd)
            return ca, cb

        def drain(sv, dv, ss, sd, ci):
            eb = base + ci * _CHUNK
            pltpu.make_async_copy(src_hbm.at[pl.ds(eb, _CHUNK)], sv,
                                  ss).wait()
            pltpu.make_async_copy(dst_hbm.at[pl.ds(eb, _CHUNK)], dv,
                                  sd).wait()

        pltpu.sync_copy(zeros_hbm, rows_a)
        zc = _CHUNK
        for j in range((rows_per_tile + zc - 1) // zc):
            m = min(zc, rows_per_tile - j * zc)
            pltpu.sync_copy(rows_a.at[pl.ds(0, m)],
                            acc_sh.at[pl.ds(r0 + j * zc, m)])
        plsc.subcore_barrier()

        def pair(c0, prefetch):
            drain(src_a, dst_a, sas, sad, c0)
            drain(src_b, dst_b, sbs, sbd, c0 + 1)
            ga = pltpu.async_copy(table_hbm.at[src_a], rows_a, sga)
            gb = pltpu.async_copy(table_hbm.at[src_b], rows_b, sgb)
            if prefetch:
                issue(src_a, dst_a, sas, sad, c0 + 2)
                issue(src_b, dst_b, sbs, sbd, c0 + 3)
            ga.wait()
            gb.wait()
            s_a = pltpu.async_copy(rows_a, acc_sh.at[dst_a], ssa, add=True)
            s_b = pltpu.async_copy(rows_b, acc_sh.at[dst_b], ssb, add=True)
            s_a.wait()
            s_b.wait()

        issue(src_a, dst_a, sas, sad, 0)
        issue(src_b, dst_b, sbs, sbd, 1)

        @pl.loop(0, nfull // 2 - 1)
        def body(gi):
            pair(2 * gi, True)

        pair(nfull - 2, False)

        if tail:
            eb = base + nfull * _CHUNK
            pltpu.sync_copy(src_hbm.at[pl.ds(eb, tail)], srct_v)
            pltpu.sync_copy(dst_hbm.at[pl.ds(eb, tail)], dstt_v)
            pltpu.async_copy(table_hbm.at[srct_v], rowst_v, sga).wait()
            pltpu.sync_copy(rowst_v, acc_sh.at[dstt_v], add=True)
        plsc.subcore_barrier()
        pltpu.sync_copy(acc_sh.at[pl.ds(r0, rows_per_tile)],
                        out_hbm.at[cid, pl.ds(r0, rows_per_tile)])

    return k


@functools.partial(jax.jit, static_argnums=())
def _sc_scatter(table, src, dst):
    n, d = table.shape
    zero_rows = jnp.zeros((_CHUNK, d), jnp.float32)
    k = _sc_scatter_build(n, d, src.shape[0])
    return k(table, src, dst, zero_rows)


_BLK = 1000


def _dinv_from_degp(degp):
    deg = degp[0, :, 0] + degp[1, :, 0] + 1.0
    return lax.rsqrt(deg)


def _pre_body(degp_ref, x_ref, w1_ref, b1_ref, h_ref, hs_ref):
    dinv = _dinv_from_degp(degp_ref[...])
    h = jnp.maximum(
        jnp.dot(x_ref[...], w1_ref[...],
                preferred_element_type=jnp.float32) + b1_ref[...], 0.0)
    h_ref[...] = h
    hs_ref[...] = h * dinv[:, None]


def _pre(degp, x, w1, b1):
    n, dft = x.shape
    hid = w1.shape[1]
    grid = n // _BLK
    return pl.pallas_call(
        _pre_body,
        grid=(grid,),
        in_specs=[
            pl.BlockSpec((_NC, _BLK, 128), lambda i: (0, i, 0)),
            pl.BlockSpec((_BLK, dft), lambda i: (i, 0)),
            pl.BlockSpec((dft, hid), lambda i: (0, 0)),
            pl.BlockSpec((1, hid), lambda i: (0, 0)),
        ],
        out_specs=[
            pl.BlockSpec((_BLK, hid), lambda i: (i, 0)),
            pl.BlockSpec((_BLK, hid), lambda i: (i, 0)),
        ],
        out_shape=[
            jax.ShapeDtypeStruct((n, hid), jnp.float32),
            jax.ShapeDtypeStruct((n, hid), jnp.float32),
        ],
    )(degp, x, w1, b1.reshape(1, hid))


def _layer_body(beta, degp_ref, part_ref, curs_ref, x0_ref, w_ref,
                cur_ref, curs_out_ref):
    dinv = _dinv_from_degp(degp_ref[...])
    s = part_ref[0] + part_ref[1] + curs_ref[...]
    agg = s * dinv[:, None]
    out = (1.0 - _ALPHA) * agg + _ALPHA * x0_ref[...]
    out = (1.0 - beta) * out + beta * jnp.dot(
        out, w_ref[...], preferred_element_type=jnp.float32)
    cur = jnp.maximum(out, 0.0)
    cur_ref[...] = cur
    curs_out_ref[...] = cur * dinv[:, None]


def _layer(beta, degp, part, curs, x0, w):
    n, hid = x0.shape
    grid = n // _BLK
    return pl.pallas_call(
        functools.partial(_layer_body, beta),
        grid=(grid,),
        in_specs=[
            pl.BlockSpec((_NC, _BLK, 128), lambda i: (0, i, 0)),
            pl.BlockSpec((_NC, _BLK, hid), lambda i: (0, i, 0)),
            pl.BlockSpec((_BLK, hid), lambda i: (i, 0)),
            pl.BlockSpec((_BLK, hid), lambda i: (i, 0)),
            pl.BlockSpec((hid, hid), lambda i: (0, 0)),
        ],
        out_specs=[
            pl.BlockSpec((_BLK, hid), lambda i: (i, 0)),
            pl.BlockSpec((_BLK, hid), lambda i: (i, 0)),
        ],
        out_shape=[
            jax.ShapeDtypeStruct((n, hid), jnp.float32),
            jax.ShapeDtypeStruct((n, hid), jnp.float32),
        ],
    )(degp, part, curs, x0, w)


def _final_body(cur_ref, w2_ref, b2_ref, o_ref):
    logits = jnp.dot(cur_ref[...], w2_ref[...],
                     preferred_element_type=jnp.float32) + b2_ref[...]
    m = jnp.max(logits, axis=1, keepdims=True)
    shifted = logits - m
    lse = jnp.log(jnp.sum(jnp.exp(shifted), axis=1, keepdims=True))
    o_ref[...] = shifted - lse


def _final(cur, w2, b2):
    n, hid = cur.shape
    nc = w2.shape[1]
    grid = n // _BLK
    return pl.pallas_call(
        _final_body,
        grid=(grid,),
        in_specs=[
            pl.BlockSpec((_BLK, hid), lambda i: (i, 0)),
            pl.BlockSpec((hid, nc), lambda i: (0, 0)),
            pl.BlockSpec((1, nc), lambda i: (0, 0)),
        ],
        out_specs=pl.BlockSpec((_BLK, nc), lambda i: (i, 0)),
        out_shape=jax.ShapeDtypeStruct((n, nc), jnp.float32),
    )(cur, w2, b2.reshape(1, nc))


def kernel(x, edge_index, y, W1, b1, Ws, W2, b2):
    n = x.shape[0]
    src = edge_index[0]
    dst = edge_index[1]
    degp = _sc_degree(dst, n)
    h, curs = _pre(degp, x, W1, b1)
    x0 = h
    cur = h
    for i in range(Ws.shape[0]):
        part = _sc_scatter(curs, src, dst)
        beta = float(np.log(_THETA / (i + 1) + 1.0))
        cur, curs = _layer(beta, degp, part, curs, x0, Ws[i])
    return _final(cur, W2, b2)
